# baseline (speedup 1.0000x reference)
import functools

import jax
import jax.numpy as jnp
from jax import lax
from jax.experimental import pallas as pl
from jax.experimental.pallas import tpu as pltpu
from jax.experimental.pallas import tpu_sc as plsc

NUM_CORES = 2
NUM_SUBCORES = 16
LANES = 16
NUM_WORKERS = NUM_CORES * NUM_SUBCORES

IDX_HALF = 8192
OUT_CHUNK = 4096


def _make_gather(n_trees, n_leaves, batch, n_classes):
  n_pairs = n_trees * n_classes
  pairs_per_worker = n_pairs // NUM_WORKERS
  assert n_pairs == pairs_per_worker * NUM_WORKERS
  assert batch == 2 * IDX_HALF and IDX_HALF == 2 * OUT_CHUNK

  mesh = plsc.VectorSubcoreMesh(
      core_axis_name="c", subcore_axis_name="s",
      num_cores=NUM_CORES, num_subcores=NUM_SUBCORES)

  @functools.partial(
      pl.kernel,
      mesh=mesh,
      compiler_params=pltpu.CompilerParams(
          use_tc_tiling_on_sc=True, needs_layout_passes=False),
      out_type=jax.ShapeDtypeStruct((n_trees, n_classes, batch), jnp.float32),
      scratch_types=[
          pltpu.VMEM((n_leaves,), jnp.float32),
          pltpu.VMEM((2, IDX_HALF), jnp.int32),
          pltpu.VMEM((2, OUT_CHUNK), jnp.float32),
          pltpu.SemaphoreType.DMA,
      ]
      + [pltpu.SemaphoreType.DMA] * 2
      + [pltpu.SemaphoreType.DMA] * 2,
  )
  def gather_kernel(table, idx3, out, row_v, idx_v, out_v, row_sem,
                    idx_sem0, idx_sem1, out_sem0, out_sem1):
    idx_sem = (idx_sem0, idx_sem1)
    out_sem = (out_sem0, out_sem1)
    wid = lax.axis_index("s") * NUM_CORES + lax.axis_index("c")
    first_pair = wid * pairs_per_worker

    def tree_cls(pair):
      return pair // n_classes, pair % n_classes

    tree0, _ = tree_cls(first_pair)
    for i in range(2):
      pltpu.async_copy(idx3.at[tree0, pl.ds(i * IDX_HALF, IDX_HALF)],
                       idx_v.at[i], idx_sem[i])

    @pl.loop(0, pairs_per_worker)
    def _pair(p):
      pair = first_pair + p
      tree, cls = tree_cls(pair)
      pltpu.async_copy(table.at[tree, cls], row_v, row_sem).wait()

      reloaded = jnp.logical_or(p == 0, cls == 0)
      for i in range(2):
        @pl.when(reloaded)
        def _():
          pltpu.make_async_copy(
              idx3.at[tree, pl.ds(i * IDX_HALF, IDX_HALF)],
              idx_v.at[i], idx_sem[i]).wait()
        for c in range(2):
          boff = i * IDX_HALF + c * OUT_CHUNK
          if i == 0:
            @pl.when(p != 0)
            def _():
              pltpu.make_async_copy(
                  out_v.at[c], out.at[tree, cls, pl.ds(0, OUT_CHUNK)],
                  out_sem[c]).wait()
          else:
            pltpu.make_async_copy(
                out_v.at[c], out.at[tree, cls, pl.ds(0, OUT_CHUNK)],
                out_sem[c]).wait()

          @plsc.parallel_loop(0, OUT_CHUNK // LANES, unroll=8)
          def _vec(k):
            sl = pl.ds(k * LANES, LANES)
            isl = pl.ds(c * OUT_CHUNK + k * LANES, LANES)
            out_v[c, sl] = plsc.load_gather(row_v, [idx_v[i, isl]])

          pltpu.async_copy(out_v.at[c], out.at[tree, cls, pl.ds(boff, OUT_CHUNK)],
                           out_sem[c])

        @pl.when(jnp.logical_and(p != pairs_per_worker - 1,
                                 cls == n_classes - 1))
        def _():
          ntree, _ = tree_cls(pair + 1)
          pltpu.async_copy(idx3.at[ntree, pl.ds(i * IDX_HALF, IDX_HALF)],
                           idx_v.at[i], idx_sem[i])

    for c in range(2):
      pltpu.make_async_copy(
          out_v.at[c], out.at[0, 0, pl.ds(0, OUT_CHUNK)], out_sem[c]).wait()

  return gather_kernel


def kernel(leafs, idx):
  n_trees, n_leaves, n_classes = leafs.shape
  batch = idx.shape[1]
  table = jnp.swapaxes(leafs, 1, 2)
  idx3 = idx.astype(jnp.int32)
  out = _make_gather(n_trees, n_leaves, batch, n_classes)(table, idx3)
  return jnp.swapaxes(out, 1, 2)
